# Initial kernel scaffold; baseline (speedup 1.0000x reference)
import jax
import jax.numpy as jnp
from jax.experimental import pallas as pl


def kernel(input_ids, emb, Wr, br, W1, b1, W2, b2, Wlm, blm):
    raise NotImplementedError("write your pallas kernel here")


# speedup vs baseline: 2.4782x; 2.4782x over previous
import functools

import jax
import jax.numpy as jnp
from jax import lax
from jax.experimental import pallas as pl
from jax.experimental.pallas import tpu as pltpu, tpu_sc as plsc

H = 1024
E = 8
K = 2
V = 32000
F = 4096
S = 2048

_NEG_INF = float("-inf")


def _sc_gather(table, idx, n_rows, d):
    info = plsc.get_sparse_core_info()
    nw = info.num_cores * info.num_subcores
    per_w = n_rows // nw
    mesh = plsc.VectorSubcoreMesh(core_axis_name="c", subcore_axis_name="s")

    @functools.partial(
        pl.kernel,
        mesh=mesh,
        out_type=jax.ShapeDtypeStruct((n_rows, d), jnp.float32),
        scratch_types=[
            pltpu.VMEM((per_w,), jnp.int32),
            pltpu.VMEM((per_w, d), jnp.float32),
            pltpu.SemaphoreType.DMA,
        ],
    )
    def k(table_hbm, idx_hbm, out_hbm, idx_v, rows_v, sem):
        wid = lax.axis_index("s") * info.num_cores + lax.axis_index("c")
        base = wid * per_w
        pltpu.sync_copy(idx_hbm.at[pl.ds(base, per_w)], idx_v)
        pltpu.async_copy(table_hbm.at[idx_v], rows_v, sem).wait()
        pltpu.sync_copy(rows_v, out_hbm.at[pl.ds(base, per_w)])

    return k(table, idx)


def _router_body(h_ref, wr_ref, br_ref, rw_ref, sel_ref):
    logits = jnp.dot(h_ref[...], wr_ref[...], preferred_element_type=jnp.float32)
    logits = logits + br_ref[...][None, :]
    col = lax.broadcasted_iota(jnp.int32, logits.shape, 1)
    valid = col < E
    logits = jnp.where(valid, logits, _NEG_INF)
    m1 = jnp.max(logits, axis=1, keepdims=True)
    a1 = jnp.min(jnp.where(logits == m1, col, logits.shape[1]), axis=1, keepdims=True)
    l2 = jnp.where(col == a1, _NEG_INF, logits)
    m2 = jnp.max(l2, axis=1, keepdims=True)
    a2 = jnp.min(jnp.where(l2 == m2, col, logits.shape[1]), axis=1, keepdims=True)
    e2 = jnp.exp(m2 - m1)
    denom = 1.0 + e2
    w1 = 1.0 / denom
    w2 = e2 / denom
    lane = lax.broadcasted_iota(jnp.int32, rw_ref.shape, 1)
    rw_ref[...] = jnp.where(lane == 0, w1, jnp.where(lane == 1, w2, 0.0))
    sel_ref[...] = jnp.where(lane == 0, a1, jnp.where(lane == 1, a2, 0))


def _router(hidden, wr_pad, br_pad):
    rw, sel = pl.pallas_call(
        _router_body,
        out_shape=(
            jax.ShapeDtypeStruct((S, 128), jnp.float32),
            jax.ShapeDtypeStruct((S, 128), jnp.int32),
        ),
    )(hidden, wr_pad, br_pad)
    return rw[:, :K], sel[:, :K]


_FC = 1024
_NFC = F // _FC


def _ffn_body(sel_ref, x_ref, w1_ref, b1_ref, w2_ref, b2_ref, out_ref, h1_ref):
    e = pl.program_id(0)
    fc = pl.program_id(1)

    @pl.when(jnp.logical_and(e == 0, fc == 0))
    def _():
        out_ref[...] = jnp.zeros_like(out_ref)

    x = x_ref[...]
    h1 = jnp.dot(x, w1_ref[0], preferred_element_type=jnp.float32)
    h1 = h1 + b1_ref[0]
    h1 = h1 * 0.5 * (1.0 + lax.erf(h1 * (2.0 ** -0.5)))
    h2 = jnp.dot(h1, w2_ref[0], preferred_element_type=jnp.float32)
    del h1_ref
    mask = jnp.any(sel_ref[...] == e, axis=1, keepdims=True).astype(jnp.float32)
    bias = jnp.where(fc == 0, 1.0, 0.0)
    h2 = h2 + bias * b2_ref[0]
    out_ref[...] += mask * h2


def _ffn_dense(hidden, sel, w1, b1, w2, b2):
    return pl.pallas_call(
        _ffn_body,
        grid=(E, _NFC),
        in_specs=[
            pl.BlockSpec((S, K), lambda e, fc: (0, 0)),
            pl.BlockSpec((S, H), lambda e, fc: (0, 0)),
            pl.BlockSpec((1, H, _FC), lambda e, fc: (e, 0, fc)),
            pl.BlockSpec((1, 1, _FC), lambda e, fc: (e, 0, fc)),
            pl.BlockSpec((1, _FC, H), lambda e, fc: (e, fc, 0)),
            pl.BlockSpec((1, 1, H), lambda e, fc: (e, 0, 0)),
        ],
        out_specs=pl.BlockSpec((S, H), lambda e, fc: (0, 0)),
        out_shape=jax.ShapeDtypeStruct((S, H), jnp.float32),
        scratch_shapes=[pltpu.VMEM((S, _FC), jnp.float32)],
    )(sel, hidden, w1, b1.reshape(E, 1, F), w2, b2.reshape(E, 1, H))


_VC = 1280
_NVC = V // _VC


def _lm_body(h_ref, w_ref, b_ref, out_ref):
    out_ref[...] = (
        jnp.dot(h_ref[...], w_ref[...], preferred_element_type=jnp.float32)
        + b_ref[...]
    )


def _lm_head(hidden_out, wlm, blm):
    return pl.pallas_call(
        _lm_body,
        grid=(_NVC,),
        in_specs=[
            pl.BlockSpec((S, H), lambda v: (0, 0)),
            pl.BlockSpec((H, _VC), lambda v: (0, v)),
            pl.BlockSpec((1, _VC), lambda v: (0, v)),
        ],
        out_specs=pl.BlockSpec((S, _VC), lambda v: (0, v)),
        out_shape=jax.ShapeDtypeStruct((S, V), jnp.float32),
    )(hidden_out, wlm, blm.reshape(1, V))


def kernel(input_ids, emb, Wr, br, W1, b1, W2, b2, Wlm, blm):
    batch, seq = input_ids.shape
    ids = input_ids.reshape(-1).astype(jnp.int32)

    hidden = _sc_gather(emb, ids, S, H)

    wr_pad = jnp.zeros((H, 128), jnp.float32).at[:, :E].set(Wr)
    br_pad = jnp.zeros((128,), jnp.float32).at[:E].set(br)
    rw, sel = _router(hidden, wr_pad, br_pad)

    hidden_out = _ffn_dense(hidden, sel, W1, b1, W2, b2)

    logits = _lm_head(hidden_out, Wlm, blm)
    return (logits.reshape(batch, seq, V), rw, sel)
